# Initial kernel scaffold; baseline (speedup 1.0000x reference)
import jax
import jax.numpy as jnp
from jax.experimental import pallas as pl


def kernel(x_cont, buckets, flop_embed, turn_embed, river_embed, w1, b1, w2, b2, w3, b3):
    raise NotImplementedError("write your pallas kernel here")


# speedup vs baseline: 2.1674x; 2.1674x over previous
import functools

import jax
import jax.numpy as jnp
from jax import lax
from jax.experimental import pallas as pl
from jax.experimental.pallas import tpu as pltpu
from jax.experimental.pallas import tpu_sc as plsc

N = 16384
EMBED_DIM = 32
CONT_DIM = 242
HID = 256
NUM_ACTIONS = 5
NUM_STREETS = 3

_info = plsc.get_sparse_core_info()
_NC = _info.num_cores
_NS = _info.num_subcores
_NW = _NC * _NS
_BPW = N // _NW

_sc_mesh = plsc.VectorSubcoreMesh(core_axis_name="c", subcore_axis_name="s")


_e_type = jax.ShapeDtypeStruct((N, EMBED_DIM), jnp.float32)


@functools.partial(
    pl.kernel,
    mesh=_sc_mesh,
    out_type=(_e_type, _e_type, _e_type),
    scratch_types=[
        pltpu.VMEM((_BPW,), jnp.int32),
        pltpu.VMEM((_BPW, EMBED_DIM), jnp.float32),
        pltpu.SemaphoreType.DMA,
    ],
    compiler_params=pltpu.CompilerParams(use_tc_tiling_on_sc=False),
)
def _gather3(b0, b1_idx, b2, flop_hbm, turn_hbm, river_hbm,
             ef_hbm, et_hbm, er_hbm, idx_v, rows_v, sem):
    wid = lax.axis_index("s") * _NC + lax.axis_index("c")
    base = wid * _BPW
    for idx_hbm, tab, out_hbm in ((b0, flop_hbm, ef_hbm),
                                  (b1_idx, turn_hbm, et_hbm),
                                  (b2, river_hbm, er_hbm)):
        pltpu.sync_copy(idx_hbm.at[pl.ds(base, _BPW)], idx_v)
        pltpu.async_copy(tab.at[idx_v], rows_v, sem).wait()
        pltpu.sync_copy(rows_v, out_hbm.at[pl.ds(base, _BPW)])


def _mlp_body(x_ref, ef_ref, et_ref, er_ref, w1c_ref, w1f_ref, w1t_ref,
              w1r_ref, b1_ref, w2_ref, b2_ref, w3_ref, b3_ref, o_ref):
    h = jnp.dot(x_ref[...], w1c_ref[...], preferred_element_type=jnp.float32)
    h += jnp.dot(ef_ref[...], w1f_ref[...], preferred_element_type=jnp.float32)
    h += jnp.dot(et_ref[...], w1t_ref[...], preferred_element_type=jnp.float32)
    h += jnp.dot(er_ref[...], w1r_ref[...], preferred_element_type=jnp.float32)
    h = jnp.maximum(h + b1_ref[...], 0.0)
    h = jnp.maximum(
        jnp.dot(h, w2_ref[...], preferred_element_type=jnp.float32) + b2_ref[...],
        0.0)
    o_ref[...] = jnp.dot(h, w3_ref[...], preferred_element_type=jnp.float32) + b3_ref[...]


_BT = 1024


@jax.jit
def kernel(x_cont, buckets, flop_embed, turn_embed, river_embed,
           w1, b1, w2, b2, w3, b3):
    ef, et, er = _gather3(buckets[:, 0], buckets[:, 1], buckets[:, 2],
                          flop_embed, turn_embed, river_embed)

    w1c = w1[:CONT_DIM]
    w1f = w1[CONT_DIM:CONT_DIM + EMBED_DIM]
    w1t = w1[CONT_DIM + EMBED_DIM:CONT_DIM + 2 * EMBED_DIM]
    w1r = w1[CONT_DIM + 2 * EMBED_DIM:]

    grid = (N // _BT,)
    full = lambda i: (0, 0)
    batch_blk = lambda d: pl.BlockSpec((_BT, d), lambda i: (i, 0))
    out = pl.pallas_call(
        _mlp_body,
        grid=grid,
        in_specs=[
            batch_blk(CONT_DIM),
            batch_blk(EMBED_DIM),
            batch_blk(EMBED_DIM),
            batch_blk(EMBED_DIM),
            pl.BlockSpec((CONT_DIM, HID), full),
            pl.BlockSpec((EMBED_DIM, HID), full),
            pl.BlockSpec((EMBED_DIM, HID), full),
            pl.BlockSpec((EMBED_DIM, HID), full),
            pl.BlockSpec((1, HID), full),
            pl.BlockSpec((HID, HID), full),
            pl.BlockSpec((1, HID), full),
            pl.BlockSpec((HID, NUM_ACTIONS), full),
            pl.BlockSpec((1, NUM_ACTIONS), full),
        ],
        out_specs=pl.BlockSpec((_BT, NUM_ACTIONS), lambda i: (i, 0)),
        out_shape=jax.ShapeDtypeStruct((N, NUM_ACTIONS), jnp.float32),
        compiler_params=pltpu.CompilerParams(
            dimension_semantics=("parallel",)),
    )(x_cont, ef, et, er, w1c, w1f, w1t, w1r,
      b1.reshape(1, HID), w2, b2.reshape(1, HID), w3,
      b3.reshape(1, NUM_ACTIONS))
    return out


# speedup vs baseline: 2.2087x; 1.0191x over previous
import functools

import jax
import jax.numpy as jnp
from jax import lax
from jax.experimental import pallas as pl
from jax.experimental.pallas import tpu as pltpu
from jax.experimental.pallas import tpu_sc as plsc

N = 16384
EMBED_DIM = 32
CONT_DIM = 242
HID = 256
NUM_ACTIONS = 5
NUM_STREETS = 3

_info = plsc.get_sparse_core_info()
_NC = _info.num_cores
_NS = _info.num_subcores
_NW = _NC * _NS
_BPW = N // _NW

_sc_mesh = plsc.VectorSubcoreMesh(core_axis_name="c", subcore_axis_name="s")


_e_type = jax.ShapeDtypeStruct((N, EMBED_DIM), jnp.float32)


@functools.partial(
    pl.kernel,
    mesh=_sc_mesh,
    out_type=(_e_type, _e_type, _e_type),
    scratch_types=[
        pltpu.VMEM((_BPW,), jnp.int32),
        pltpu.VMEM((_BPW, EMBED_DIM), jnp.float32),
        pltpu.SemaphoreType.DMA,
    ],
    compiler_params=pltpu.CompilerParams(use_tc_tiling_on_sc=False),
)
def _gather3(b0, b1_idx, b2, flop_hbm, turn_hbm, river_hbm,
             ef_hbm, et_hbm, er_hbm, idx_v, rows_v, sem):
    wid = lax.axis_index("s") * _NC + lax.axis_index("c")
    base = wid * _BPW
    for idx_hbm, tab, out_hbm in ((b0, flop_hbm, ef_hbm),
                                  (b1_idx, turn_hbm, et_hbm),
                                  (b2, river_hbm, er_hbm)):
        pltpu.sync_copy(idx_hbm.at[pl.ds(base, _BPW)], idx_v)
        pltpu.async_copy(tab.at[idx_v], rows_v, sem).wait()
        pltpu.sync_copy(rows_v, out_hbm.at[pl.ds(base, _BPW)])


def _mlp_body(x_ref, ef_ref, et_ref, er_ref, w1c_ref, w1e_ref,
              b1_ref, w2_ref, b2_ref, w3_ref, b3_ref, o_ref):
    h = jnp.dot(x_ref[...], w1c_ref[...], preferred_element_type=jnp.float32)
    e = jnp.concatenate([ef_ref[...], et_ref[...], er_ref[...]], axis=1)
    h += jnp.dot(e, w1e_ref[...], preferred_element_type=jnp.float32)
    h = jnp.maximum(h + b1_ref[...], 0.0)
    h = jnp.maximum(
        jnp.dot(h, w2_ref[...], preferred_element_type=jnp.float32) + b2_ref[...],
        0.0)
    o_ref[...] = jnp.dot(h, w3_ref[...], preferred_element_type=jnp.float32) + b3_ref[...]


_BT = 1024


@jax.jit
def kernel(x_cont, buckets, flop_embed, turn_embed, river_embed,
           w1, b1, w2, b2, w3, b3):
    ef, et, er = _gather3(buckets[:, 0], buckets[:, 1], buckets[:, 2],
                          flop_embed, turn_embed, river_embed)

    w1c = w1[:CONT_DIM]
    w1e = w1[CONT_DIM:]

    grid = (N // _BT,)
    full = lambda i: (0, 0)
    batch_blk = lambda d: pl.BlockSpec((_BT, d), lambda i: (i, 0))
    out = pl.pallas_call(
        _mlp_body,
        grid=grid,
        in_specs=[
            batch_blk(CONT_DIM),
            batch_blk(EMBED_DIM),
            batch_blk(EMBED_DIM),
            batch_blk(EMBED_DIM),
            pl.BlockSpec((CONT_DIM, HID), full),
            pl.BlockSpec((NUM_STREETS * EMBED_DIM, HID), full),
            pl.BlockSpec((1, HID), full),
            pl.BlockSpec((HID, HID), full),
            pl.BlockSpec((1, HID), full),
            pl.BlockSpec((HID, NUM_ACTIONS), full),
            pl.BlockSpec((1, NUM_ACTIONS), full),
        ],
        out_specs=pl.BlockSpec((_BT, NUM_ACTIONS), lambda i: (i, 0)),
        out_shape=jax.ShapeDtypeStruct((N, NUM_ACTIONS), jnp.float32),
        compiler_params=pltpu.CompilerParams(
            dimension_semantics=("parallel",)),
    )(x_cont, ef, et, er, w1c, w1e,
      b1.reshape(1, HID), w2, b2.reshape(1, HID), w3,
      b3.reshape(1, NUM_ACTIONS))
    return out


# speedup vs baseline: 2.2140x; 1.0024x over previous
import functools

import jax
import jax.numpy as jnp
from jax import lax
from jax.experimental import pallas as pl
from jax.experimental.pallas import tpu as pltpu
from jax.experimental.pallas import tpu_sc as plsc

N = 16384
EMBED_DIM = 32
CONT_DIM = 242
HID = 256
NUM_ACTIONS = 5
NUM_STREETS = 3

_info = plsc.get_sparse_core_info()
_NC = _info.num_cores
_NS = _info.num_subcores
_NW = _NC * _NS
_BPW = N // _NW

_sc_mesh = plsc.VectorSubcoreMesh(core_axis_name="c", subcore_axis_name="s")


_e_type = jax.ShapeDtypeStruct((N, EMBED_DIM), jnp.float32)


@functools.partial(
    pl.kernel,
    mesh=_sc_mesh,
    out_type=(_e_type, _e_type, _e_type),
    scratch_types=[
        pltpu.VMEM((_BPW,), jnp.int32),
        pltpu.VMEM((_BPW, EMBED_DIM), jnp.float32),
        pltpu.SemaphoreType.DMA,
    ],
    compiler_params=pltpu.CompilerParams(use_tc_tiling_on_sc=False),
)
def _gather3(b0, b1_idx, b2, flop_hbm, turn_hbm, river_hbm,
             ef_hbm, et_hbm, er_hbm, idx_v, rows_v, sem):
    wid = lax.axis_index("s") * _NC + lax.axis_index("c")
    base = wid * _BPW
    for idx_hbm, tab, out_hbm in ((b0, flop_hbm, ef_hbm),
                                  (b1_idx, turn_hbm, et_hbm),
                                  (b2, river_hbm, er_hbm)):
        pltpu.sync_copy(idx_hbm.at[pl.ds(base, _BPW)], idx_v)
        pltpu.async_copy(tab.at[idx_v], rows_v, sem).wait()
        pltpu.sync_copy(rows_v, out_hbm.at[pl.ds(base, _BPW)])


def _mlp_body(x_ref, ef_ref, et_ref, er_ref, w1c_ref, w1e_ref,
              b1_ref, w2_ref, b2_ref, w3_ref, b3_ref, o_ref):
    bf = jnp.bfloat16
    h = jnp.dot(x_ref[...].astype(bf), w1c_ref[...].astype(bf),
                preferred_element_type=jnp.float32)
    e = jnp.concatenate([ef_ref[...], et_ref[...], er_ref[...]], axis=1)
    h += jnp.dot(e.astype(bf), w1e_ref[...].astype(bf),
                 preferred_element_type=jnp.float32)
    h = jnp.maximum(h + b1_ref[...], 0.0)
    h = jnp.maximum(
        jnp.dot(h.astype(bf), w2_ref[...].astype(bf),
                preferred_element_type=jnp.float32) + b2_ref[...],
        0.0)
    o_ref[...] = jnp.dot(h.astype(bf), w3_ref[...].astype(bf),
                         preferred_element_type=jnp.float32) + b3_ref[...]


_BT = 1024


@jax.jit
def kernel(x_cont, buckets, flop_embed, turn_embed, river_embed,
           w1, b1, w2, b2, w3, b3):
    ef, et, er = _gather3(buckets[:, 0], buckets[:, 1], buckets[:, 2],
                          flop_embed, turn_embed, river_embed)

    w1c = w1[:CONT_DIM]
    w1e = w1[CONT_DIM:]

    grid = (N // _BT,)
    full = lambda i: (0, 0)
    batch_blk = lambda d: pl.BlockSpec((_BT, d), lambda i: (i, 0))
    out = pl.pallas_call(
        _mlp_body,
        grid=grid,
        in_specs=[
            batch_blk(CONT_DIM),
            batch_blk(EMBED_DIM),
            batch_blk(EMBED_DIM),
            batch_blk(EMBED_DIM),
            pl.BlockSpec((CONT_DIM, HID), full),
            pl.BlockSpec((NUM_STREETS * EMBED_DIM, HID), full),
            pl.BlockSpec((1, HID), full),
            pl.BlockSpec((HID, HID), full),
            pl.BlockSpec((1, HID), full),
            pl.BlockSpec((HID, NUM_ACTIONS), full),
            pl.BlockSpec((1, NUM_ACTIONS), full),
        ],
        out_specs=pl.BlockSpec((_BT, NUM_ACTIONS), lambda i: (i, 0)),
        out_shape=jax.ShapeDtypeStruct((N, NUM_ACTIONS), jnp.float32),
        compiler_params=pltpu.CompilerParams(
            dimension_semantics=("parallel",)),
    )(x_cont, ef, et, er, w1c, w1e,
      b1.reshape(1, HID), w2, b2.reshape(1, HID), w3,
      b3.reshape(1, NUM_ACTIONS))
    return out


# speedup vs baseline: 2.3263x; 1.0507x over previous
import functools

import jax
import jax.numpy as jnp
from jax import lax
from jax.experimental import pallas as pl
from jax.experimental.pallas import tpu as pltpu
from jax.experimental.pallas import tpu_sc as plsc

N = 16384
EMBED_DIM = 32
CONT_DIM = 242
HID = 256
NUM_ACTIONS = 5
NUM_STREETS = 3

_info = plsc.get_sparse_core_info()
_NC = _info.num_cores
_NS = _info.num_subcores
_NW = _NC * _NS
_BPW = N // _NW

_sc_mesh = plsc.VectorSubcoreMesh(core_axis_name="c", subcore_axis_name="s")


_e_type = jax.ShapeDtypeStruct((N, EMBED_DIM), jnp.float32)


@functools.partial(
    pl.kernel,
    mesh=_sc_mesh,
    out_type=(_e_type, _e_type, _e_type),
    scratch_types=[
        pltpu.VMEM((_BPW,), jnp.int32),
        pltpu.VMEM((_BPW, EMBED_DIM), jnp.float32),
        pltpu.SemaphoreType.DMA,
    ],
    compiler_params=pltpu.CompilerParams(use_tc_tiling_on_sc=False),
)
def _gather3(b0, b1_idx, b2, flop_hbm, turn_hbm, river_hbm,
             ef_hbm, et_hbm, er_hbm, idx_v, rows_v, sem):
    wid = lax.axis_index("s") * _NC + lax.axis_index("c")
    base = wid * _BPW
    for idx_hbm, tab, out_hbm in ((b0, flop_hbm, ef_hbm),
                                  (b1_idx, turn_hbm, et_hbm),
                                  (b2, river_hbm, er_hbm)):
        pltpu.sync_copy(idx_hbm.at[pl.ds(base, _BPW)], idx_v)
        pltpu.async_copy(tab.at[idx_v], rows_v, sem).wait()
        pltpu.sync_copy(rows_v, out_hbm.at[pl.ds(base, _BPW)])


def _mlp_body(x_ref, ef_ref, et_ref, er_ref, w1c_ref, w1e_ref,
              b1_ref, w2_ref, b2_ref, w3_ref, b3_ref, o_ref):
    bf = jnp.bfloat16
    h = jnp.dot(x_ref[...].astype(bf), w1c_ref[...].astype(bf),
                preferred_element_type=jnp.float32)
    e = jnp.concatenate([ef_ref[...], et_ref[...], er_ref[...]], axis=1)
    h += jnp.dot(e.astype(bf), w1e_ref[...].astype(bf),
                 preferred_element_type=jnp.float32)
    h = jnp.maximum(h + b1_ref[...], 0.0)
    h = jnp.maximum(
        jnp.dot(h.astype(bf), w2_ref[...].astype(bf),
                preferred_element_type=jnp.float32) + b2_ref[...],
        0.0)
    o_ref[...] = jnp.dot(h.astype(bf), w3_ref[...].astype(bf),
                         preferred_element_type=jnp.float32) + b3_ref[...]


_BT = 2048


@jax.jit
def kernel(x_cont, buckets, flop_embed, turn_embed, river_embed,
           w1, b1, w2, b2, w3, b3):
    ef, et, er = _gather3(buckets[:, 0], buckets[:, 1], buckets[:, 2],
                          flop_embed, turn_embed, river_embed)

    w1c = w1[:CONT_DIM]
    w1e = w1[CONT_DIM:]

    grid = (N // _BT,)
    full = lambda i: (0, 0)
    batch_blk = lambda d: pl.BlockSpec((_BT, d), lambda i: (i, 0))
    out = pl.pallas_call(
        _mlp_body,
        grid=grid,
        in_specs=[
            batch_blk(CONT_DIM),
            batch_blk(EMBED_DIM),
            batch_blk(EMBED_DIM),
            batch_blk(EMBED_DIM),
            pl.BlockSpec((CONT_DIM, HID), full),
            pl.BlockSpec((NUM_STREETS * EMBED_DIM, HID), full),
            pl.BlockSpec((1, HID), full),
            pl.BlockSpec((HID, HID), full),
            pl.BlockSpec((1, HID), full),
            pl.BlockSpec((HID, NUM_ACTIONS), full),
            pl.BlockSpec((1, NUM_ACTIONS), full),
        ],
        out_specs=pl.BlockSpec((_BT, NUM_ACTIONS), lambda i: (i, 0)),
        out_shape=jax.ShapeDtypeStruct((N, NUM_ACTIONS), jnp.float32),
        compiler_params=pltpu.CompilerParams(
            dimension_semantics=("parallel",)),
    )(x_cont, ef, et, er, w1c, w1e,
      b1.reshape(1, HID), w2, b2.reshape(1, HID), w3,
      b3.reshape(1, NUM_ACTIONS))
    return out


# speedup vs baseline: 2.3603x; 1.0146x over previous
import functools

import jax
import jax.numpy as jnp
from jax import lax
from jax.experimental import pallas as pl
from jax.experimental.pallas import tpu as pltpu
from jax.experimental.pallas import tpu_sc as plsc

N = 16384
EMBED_DIM = 32
CONT_DIM = 242
HID = 256
NUM_ACTIONS = 5
NUM_STREETS = 3

_info = plsc.get_sparse_core_info()
_NC = _info.num_cores
_NS = _info.num_subcores
_NW = _NC * _NS
_BPW = N // _NW

_sc_mesh = plsc.VectorSubcoreMesh(core_axis_name="c", subcore_axis_name="s")


_e_type = jax.ShapeDtypeStruct((N, EMBED_DIM), jnp.float32)


@functools.partial(
    pl.kernel,
    mesh=_sc_mesh,
    out_type=(_e_type, _e_type, _e_type),
    scratch_types=[
        pltpu.VMEM((_BPW,), jnp.int32),
        pltpu.VMEM((_BPW, EMBED_DIM), jnp.float32),
        pltpu.SemaphoreType.DMA,
    ],
    compiler_params=pltpu.CompilerParams(use_tc_tiling_on_sc=False),
)
def _gather3(b0, b1_idx, b2, flop_hbm, turn_hbm, river_hbm,
             ef_hbm, et_hbm, er_hbm, idx_v, rows_v, sem):
    wid = lax.axis_index("s") * _NC + lax.axis_index("c")
    base = wid * _BPW
    for idx_hbm, tab, out_hbm in ((b0, flop_hbm, ef_hbm),
                                  (b1_idx, turn_hbm, et_hbm),
                                  (b2, river_hbm, er_hbm)):
        pltpu.sync_copy(idx_hbm.at[pl.ds(base, _BPW)], idx_v)
        pltpu.async_copy(tab.at[idx_v], rows_v, sem).wait()
        pltpu.sync_copy(rows_v, out_hbm.at[pl.ds(base, _BPW)])


def _mlp_body(x_ref, ef_ref, et_ref, er_ref, w1c_ref, w1e_ref,
              b1_ref, w2_ref, b2_ref, w3_ref, b3_ref, o_ref):
    bf = jnp.bfloat16
    h = jnp.dot(x_ref[...].astype(bf), w1c_ref[...].astype(bf),
                preferred_element_type=jnp.float32)
    e = jnp.concatenate([ef_ref[...], et_ref[...], er_ref[...]], axis=1)
    h += jnp.dot(e.astype(bf), w1e_ref[...].astype(bf),
                 preferred_element_type=jnp.float32)
    h = jnp.maximum(h + b1_ref[...], 0.0)
    h = jnp.maximum(
        jnp.dot(h.astype(bf), w2_ref[...].astype(bf),
                preferred_element_type=jnp.float32) + b2_ref[...],
        0.0)
    o_ref[...] = jnp.dot(h.astype(bf), w3_ref[...].astype(bf),
                         preferred_element_type=jnp.float32) + b3_ref[...]


_BT = 4096


@jax.jit
def kernel(x_cont, buckets, flop_embed, turn_embed, river_embed,
           w1, b1, w2, b2, w3, b3):
    ef, et, er = _gather3(buckets[:, 0], buckets[:, 1], buckets[:, 2],
                          flop_embed, turn_embed, river_embed)

    w1c = w1[:CONT_DIM]
    w1e = w1[CONT_DIM:]

    grid = (N // _BT,)
    full = lambda i: (0, 0)
    batch_blk = lambda d: pl.BlockSpec((_BT, d), lambda i: (i, 0))
    out = pl.pallas_call(
        _mlp_body,
        grid=grid,
        in_specs=[
            batch_blk(CONT_DIM),
            batch_blk(EMBED_DIM),
            batch_blk(EMBED_DIM),
            batch_blk(EMBED_DIM),
            pl.BlockSpec((CONT_DIM, HID), full),
            pl.BlockSpec((NUM_STREETS * EMBED_DIM, HID), full),
            pl.BlockSpec((1, HID), full),
            pl.BlockSpec((HID, HID), full),
            pl.BlockSpec((1, HID), full),
            pl.BlockSpec((HID, NUM_ACTIONS), full),
            pl.BlockSpec((1, NUM_ACTIONS), full),
        ],
        out_specs=pl.BlockSpec((_BT, NUM_ACTIONS), lambda i: (i, 0)),
        out_shape=jax.ShapeDtypeStruct((N, NUM_ACTIONS), jnp.float32),
        compiler_params=pltpu.CompilerParams(
            dimension_semantics=("parallel",)),
    )(x_cont, ef, et, er, w1c, w1e,
      b1.reshape(1, HID), w2, b2.reshape(1, HID), w3,
      b3.reshape(1, NUM_ACTIONS))
    return out


# speedup vs baseline: 4.3159x; 1.8285x over previous
import functools

import jax
import jax.numpy as jnp
from jax import lax
from jax.experimental import pallas as pl
from jax.experimental.pallas import tpu as pltpu
from jax.experimental.pallas import tpu_sc as plsc

N = 16384
EMBED_DIM = 32
CONT_DIM = 242
HID = 256
NUM_ACTIONS = 5
NUM_STREETS = 3
EPAD = 128

_info = plsc.get_sparse_core_info()
_NC = _info.num_cores
_NS = _info.num_subcores
_NW = _NC * _NS
_BPW = N // _NW

_sc_mesh = plsc.VectorSubcoreMesh(core_axis_name="c", subcore_axis_name="s")


@functools.partial(
    pl.kernel,
    mesh=_sc_mesh,
    out_type=jax.ShapeDtypeStruct((N, EPAD), jnp.float32),
    scratch_types=[
        pltpu.VMEM((_BPW,), jnp.int32),
        pltpu.VMEM((_BPW,), jnp.int32),
        pltpu.VMEM((_BPW,), jnp.int32),
        pltpu.VMEM((_BPW, EMBED_DIM), jnp.float32),
        pltpu.VMEM((_BPW, EMBED_DIM), jnp.float32),
        pltpu.VMEM((_BPW, EMBED_DIM), jnp.float32),
        pltpu.SemaphoreType.DMA,
        pltpu.SemaphoreType.DMA,
        pltpu.SemaphoreType.DMA,
    ],
    compiler_params=pltpu.CompilerParams(use_tc_tiling_on_sc=False),
)
def _gather3(b0, b1_idx, b2, flop_hbm, turn_hbm, river_hbm, e_hbm,
             i0, i1, i2, r0, r1, r2, s0, s1, s2):
    wid = lax.axis_index("s") * _NC + lax.axis_index("c")
    base = wid * _BPW
    pltpu.sync_copy(b0.at[pl.ds(base, _BPW)], i0)
    pltpu.sync_copy(b1_idx.at[pl.ds(base, _BPW)], i1)
    pltpu.sync_copy(b2.at[pl.ds(base, _BPW)], i2)
    c0 = pltpu.async_copy(flop_hbm.at[i0], r0, s0)
    c1 = pltpu.async_copy(turn_hbm.at[i1], r1, s1)
    c2 = pltpu.async_copy(river_hbm.at[i2], r2, s2)
    rows = pl.ds(base, _BPW)
    c0.wait()
    pltpu.sync_copy(r0, e_hbm.at[rows, pl.ds(0, EMBED_DIM)])
    pltpu.sync_copy(r0, e_hbm.at[rows, pl.ds(96, EMBED_DIM)])
    c1.wait()
    pltpu.sync_copy(r1, e_hbm.at[rows, pl.ds(32, EMBED_DIM)])
    c2.wait()
    pltpu.sync_copy(r2, e_hbm.at[rows, pl.ds(64, EMBED_DIM)])


def _mlp_body(xT_ref, e_ref, w1cT_ref, w1eT_ref, b1_ref, w2T_ref, b2_ref,
              w3T_ref, b3_ref, oT_ref):
    bf = jnp.bfloat16
    h = jnp.dot(w1cT_ref[...].astype(bf), xT_ref[...].astype(bf),
                preferred_element_type=jnp.float32)
    h += lax.dot_general(
        w1eT_ref[...].astype(bf), e_ref[...].astype(bf),
        dimension_numbers=(((1,), (1,)), ((), ())),
        preferred_element_type=jnp.float32)
    h = jnp.maximum(h + b1_ref[...], 0.0)
    h = jnp.maximum(
        jnp.dot(w2T_ref[...].astype(bf), h.astype(bf),
                preferred_element_type=jnp.float32) + b2_ref[...],
        0.0)
    oT_ref[...] = jnp.dot(w3T_ref[...].astype(bf), h.astype(bf),
                          preferred_element_type=jnp.float32) + b3_ref[...]


_BT = 4096


@jax.jit
def kernel(x_cont, buckets, flop_embed, turn_embed, river_embed,
           w1, b1, w2, b2, w3, b3):
    e = _gather3(buckets[:, 0], buckets[:, 1], buckets[:, 2],
                 flop_embed, turn_embed, river_embed)

    xT = jnp.swapaxes(x_cont, 0, 1)
    w1cT = w1[:CONT_DIM].T
    w1eT = jnp.pad(w1[CONT_DIM:].T, ((0, 0), (0, EPAD - NUM_STREETS * EMBED_DIM)))
    w2T = w2.T
    w3T = w3.T

    grid = (N // _BT,)
    full = lambda i: (0, 0)
    outT = pl.pallas_call(
        _mlp_body,
        grid=grid,
        in_specs=[
            pl.BlockSpec((CONT_DIM, _BT), lambda i: (0, i)),
            pl.BlockSpec((_BT, EPAD), lambda i: (i, 0)),
            pl.BlockSpec((HID, CONT_DIM), full),
            pl.BlockSpec((HID, EPAD), full),
            pl.BlockSpec((HID, 1), full),
            pl.BlockSpec((HID, HID), full),
            pl.BlockSpec((HID, 1), full),
            pl.BlockSpec((NUM_ACTIONS, HID), full),
            pl.BlockSpec((NUM_ACTIONS, 1), full),
        ],
        out_specs=pl.BlockSpec((NUM_ACTIONS, _BT), lambda i: (0, i)),
        out_shape=jax.ShapeDtypeStruct((NUM_ACTIONS, N), jnp.float32),
        compiler_params=pltpu.CompilerParams(
            dimension_semantics=("parallel",)),
    )(xT, e, w1cT, w1eT, b1.reshape(HID, 1), w2T, b2.reshape(HID, 1),
      w3T, b3.reshape(NUM_ACTIONS, 1))
    return jnp.swapaxes(outT, 0, 1)
